# baseline (speedup 1.0000x reference)
import functools

import jax
import jax.numpy as jnp
from jax import lax
from jax.experimental import pallas as pl
from jax.experimental.pallas import tpu as pltpu
from jax.experimental.pallas import tpu_sc as plsc

NC = 2
NCS = 2
NS = 16
G = 128


def _make_sc_agg(K0, K1, Nacc, D, Gw, with_counts):
    mesh = plsc.VectorSubcoreMesh(
        core_axis_name="c", subcore_axis_name="s", num_cores=NCS, num_subcores=NS
    )
    out_type = [jax.ShapeDtypeStruct((NCS, Nacc, D), jnp.float32)]
    if with_counts:
        out_type.append(jax.ShapeDtypeStruct((NCS, Nacc), jnp.float32))
    CH = 8
    NB = 4
    NCH0 = K0 // CH
    NCH1 = K1 // CH
    assert K0 % CH == 0 and K1 % CH == 0
    assert NCH0 % 2 == 1 and NCH1 % 2 == 1 and min(NCH0, NCH1) >= 1
    scratch = [
        pltpu.VMEM((2, CH, Gw), jnp.int32),
        pltpu.VMEM((2, CH, Gw), jnp.int32),
        pltpu.VMEM((NB, Gw, D), jnp.float32),
        pltpu.VMEM((Gw,), jnp.float32),
        pltpu.VMEM_SHARED((Nacc, D), jnp.float32),
        pltpu.VMEM_SHARED((Nacc,), jnp.float32),
        pltpu.SemaphoreType.DMA,
        pltpu.SemaphoreType.DMA,
        pltpu.SemaphoreType.DMA,
        pltpu.SemaphoreType.DMA,
    ]
    rpt = Nacc // NS

    def body(src_hbm, dst_hbm, tab_hbm, znd_hbm, zn_hbm, ones_hbm, *rest):
        if with_counts:
            part_out, cnt_out = rest[0], rest[1]
            rest = rest[2:]
        else:
            part_out = rest[0]
            cnt_out = None
            rest = rest[1:]
        sidxb, didxb, rows, onesv, acc, cacc, isem, gsem, ssem, csem = rest
        cid = lax.axis_index("c")
        sid = lax.axis_index("s")
        base = jnp.where(cid == 0, sid * K0, NS * K0 + sid * K1)
        npairs = jnp.where(cid == 0, (NCH0 - 1) // 2, (NCH1 - 1) // 2)
        clast = jnp.where(cid == 0, NCH0 - 1, NCH1 - 1)
        kmine = jnp.where(cid == 0, K0, K1)

        def stage_chunk(c1, s):
            off = pl.multiple_of(base + c1 * CH, CH)
            pltpu.async_copy(src_hbm.at[pl.ds(off, CH)], sidxb.at[s], isem)
            pltpu.async_copy(dst_hbm.at[pl.ds(off, CH)], didxb.at[s], isem)

        def wait_stage(s):
            pltpu.make_async_copy(src_hbm.at[pl.ds(0, CH)], sidxb.at[s],
                                  isem).wait()
            pltpu.make_async_copy(dst_hbm.at[pl.ds(0, CH)], didxb.at[s],
                                  isem).wait()

        def fire_gather(s, jrow, b):
            pltpu.async_copy(tab_hbm.at[sidxb.at[s, jrow]], rows.at[b], gsem)

        def wait_gather(b):
            pltpu.make_async_copy(tab_hbm.at[sidxb.at[0, 0]], rows.at[b],
                                  gsem).wait()

        def wait_scatter():
            pltpu.make_async_copy(rows.at[0], acc.at[didxb.at[0, 0]],
                                  ssem).wait()

        stage_chunk(0, 0)
        wait_stage(0)
        for b in range(NB - 1):
            fire_gather(0, b, b)
        pltpu.sync_copy(znd_hbm.at[pl.ds(sid * rpt, rpt)],
                        acc.at[pl.ds(sid * rpt, rpt)])
        if with_counts:
            pltpu.sync_copy(zn_hbm.at[pl.ds(sid * rpt, rpt)],
                            cacc.at[pl.ds(sid * rpt, rpt)])
            pltpu.sync_copy(ones_hbm, onesv)
        plsc.subcore_barrier()

        def chunk_work(c, h, is_last):
            for j in range(CH):
                b = j % NB
                wait_gather(b)
                pltpu.async_copy(rows.at[b], acc.at[didxb.at[h, j]],
                                 ssem, add=True)
                if with_counts:
                    pltpu.async_copy(onesv, cacc.at[didxb.at[h, j]],
                                     csem, add=True)
                if j == 0:
                    if is_last:
                        @pl.when(c >= 1)
                        def _():
                            wait_scatter()
                    else:
                        @pl.when(c >= 1)
                        def _():
                            wait_scatter()
                        stage_chunk(c + 1, 1 - h)
                else:
                    wait_scatter()
                jn = j + NB - 1
                if jn < CH:
                    fire_gather(h, jn, jn % NB)
                elif not is_last:
                    if jn == CH:
                        wait_stage(1 - h)
                    fire_gather(1 - h, jn - CH, jn % NB)

        def pair_body(c2, carry):
            chunk_work(2 * c2, 0, False)
            chunk_work(2 * c2 + 1, 1, False)
            return carry

        lax.fori_loop(0, npairs, pair_body, 0)
        chunk_work(clast, 0, True)
        wait_scatter()
        if with_counts:
            def cdrain(g, carry):
                pltpu.make_async_copy(onesv, cacc.at[didxb.at[0, 0]],
                                      csem).wait()
                return carry
            lax.fori_loop(0, kmine, cdrain, 0)
        plsc.subcore_barrier()
        pltpu.sync_copy(acc.at[pl.ds(sid * rpt, rpt)],
                        part_out.at[cid, pl.ds(sid * rpt, rpt)])
        if with_counts:
            pltpu.sync_copy(cacc.at[pl.ds(sid * rpt, rpt)],
                            cnt_out.at[cid, pl.ds(sid * rpt, rpt)])

    return pl.kernel(body, out_type=out_type, mesh=mesh, scratch_types=scratch)


def _tc_layer1_body(part_ref, cnt_ref, x_ref, wl_ref, bl_ref, wr_ref,
                    g_ref, b_ref, h_ref):
    agg = part_ref[0] + part_ref[1]
    c = cnt_ref[0] + cnt_ref[1]
    mean = agg * (1.0 / jnp.maximum(c, 1.0))[:, None]
    t = (jnp.dot(mean, wl_ref[...], preferred_element_type=jnp.float32)
         + bl_ref[...]
         + jnp.dot(x_ref[...], wr_ref[...], preferred_element_type=jnp.float32))
    mu = jnp.mean(t, axis=-1, keepdims=True)
    var = jnp.mean((t - mu) ** 2, axis=-1, keepdims=True)
    ln = (t - mu) * lax.rsqrt(var + 1e-5) * g_ref[...] + b_ref[...]
    h_ref[...] = jnp.maximum(ln, 0.0)


def _tc_layer2_body(part_ref, cnt_ref, h_ref, wl_ref, bl_ref, wr_ref, o_ref):
    agg = part_ref[0] + part_ref[1]
    c = cnt_ref[0] + cnt_ref[1]
    mean = agg * (1.0 / jnp.maximum(c, 1.0))[:, None]
    o_ref[...] = (jnp.dot(mean, wl_ref[...], preferred_element_type=jnp.float32)
                  + bl_ref[...]
                  + jnp.dot(h_ref[...], wr_ref[...],
                            preferred_element_type=jnp.float32))


def _make_tc_layer1(N, D, Nacc, R):
    grid = (pl.cdiv(N, R),)
    mat = pl.BlockSpec((D, D), lambda i: (0, 0))
    vec = pl.BlockSpec((1, D), lambda i: (0, 0))
    return pl.pallas_call(
        _tc_layer1_body,
        grid=grid,
        in_specs=[
            pl.BlockSpec((NCS, R, D), lambda i: (0, i, 0)),
            pl.BlockSpec((NCS, R), lambda i: (0, i)),
            pl.BlockSpec((R, D), lambda i: (i, 0)),
            mat, vec, mat, vec, vec,
        ],
        out_specs=pl.BlockSpec((R, D), lambda i: (i, 0)),
        out_shape=jax.ShapeDtypeStruct((N, D), jnp.float32),
    )


def _make_tc_layer2(N, D, Nacc, R):
    grid = (pl.cdiv(N, R),)
    mat = pl.BlockSpec((D, D), lambda i: (0, 0))
    vec = pl.BlockSpec((1, D), lambda i: (0, 0))
    return pl.pallas_call(
        _tc_layer2_body,
        grid=grid,
        in_specs=[
            pl.BlockSpec((NCS, R, D), lambda i: (0, i, 0)),
            pl.BlockSpec((NCS, R), lambda i: (0, i)),
            pl.BlockSpec((R, D), lambda i: (i, 0)),
            mat, vec, mat,
        ],
        out_specs=pl.BlockSpec((R, D), lambda i: (i, 0)),
        out_shape=jax.ShapeDtypeStruct((N, D), jnp.float32),
    )


def kernel(x, edge_index, W1l, b1l, W1r, ln_g, ln_b, W2l, b2l, W2r):
    N, D = x.shape
    E = edge_index.shape[1]
    NW = NCS * NS
    CH = 8
    Gw = 80
    K = pl.cdiv(E, NW * Gw)
    K = ((K + CH - 1) // CH) * CH
    Epad = NW * Gw * K
    K1 = CH
    K0 = 2 * K - K1
    R = 2048
    nblk = pl.cdiv(N, R)
    Nacc = max(nblk * R, N + 1)
    Nacc = ((Nacc + NS * 8 - 1) // (NS * 8)) * (NS * 8)

    src = edge_index[0]
    dst = edge_index[1]
    pad = Epad - E
    src3 = jnp.concatenate(
        [src, jnp.zeros((pad,), jnp.int32)]).reshape(Epad // Gw, Gw)
    dst3 = jnp.concatenate(
        [dst, jnp.full((pad,), N, jnp.int32)]).reshape(Epad // Gw, Gw)
    zeros_nd = jnp.zeros((Nacc, D), jnp.float32)
    zeros_n = jnp.zeros((Nacc,), jnp.float32)
    ones_g = jnp.ones((Gw,), jnp.float32)

    sc_agg_cnt = _make_sc_agg(K0, K1, Nacc, D, Gw, with_counts=True)
    sc_agg = _make_sc_agg(K0, K1, Nacc, D, Gw, with_counts=False)
    tc1 = _make_tc_layer1(N, D, Nacc, R)
    tc2 = _make_tc_layer2(N, D, Nacc, R)

    part1, cntp = sc_agg_cnt(src3, dst3, x, zeros_nd, zeros_n, ones_g)
    h = tc1(part1, cntp, x, W1l, b1l.reshape(1, D), W1r,
            ln_g.reshape(1, D), ln_b.reshape(1, D))
    (part2,) = sc_agg(src3, dst3, h, zeros_nd, zeros_n, ones_g)
    out = tc2(part2, cntp, h, W2l, b2l.reshape(1, D), W2r)
    return out


# speedup vs baseline: 1.0111x; 1.0111x over previous
import functools

import jax
import jax.numpy as jnp
from jax import lax
from jax.experimental import pallas as pl
from jax.experimental.pallas import tpu as pltpu
from jax.experimental.pallas import tpu_sc as plsc

NC = 2
NCS = 2
NS = 16
G = 128


def _make_sc_agg(K0, K1, Nacc, D, Gw, with_counts):
    mesh = plsc.VectorSubcoreMesh(
        core_axis_name="c", subcore_axis_name="s", num_cores=NCS, num_subcores=NS
    )
    out_type = [jax.ShapeDtypeStruct((NCS, Nacc, D), jnp.float32)]
    if with_counts:
        out_type.append(jax.ShapeDtypeStruct((NCS, Nacc), jnp.float32))
    CH = 8
    NB = 4
    NCH0 = K0 // CH
    NCH1 = K1 // CH
    assert K0 % CH == 0 and K1 % CH == 0
    assert NCH0 % 2 == 1 and NCH1 % 2 == 1 and min(NCH0, NCH1) >= 1
    scratch = [
        pltpu.VMEM((2, CH, Gw), jnp.int32),
        pltpu.VMEM((2, CH, Gw), jnp.int32),
        pltpu.VMEM((NB, Gw, D), jnp.float32),
        pltpu.VMEM((Gw,), jnp.float32),
        pltpu.VMEM_SHARED((Nacc, D), jnp.float32),
        pltpu.VMEM_SHARED((Nacc,), jnp.float32),
        pltpu.SemaphoreType.DMA,
        pltpu.SemaphoreType.DMA,
        pltpu.SemaphoreType.DMA,
        pltpu.SemaphoreType.DMA,
    ]
    rpt = Nacc // NS

    def body(src_hbm, dst_hbm, tab_hbm, znd_hbm, zn_hbm, ones_hbm, *rest):
        if with_counts:
            part_out, cnt_out = rest[0], rest[1]
            rest = rest[2:]
        else:
            part_out = rest[0]
            cnt_out = None
            rest = rest[1:]
        sidxb, didxb, rows, onesv, acc, cacc, isem, gsem, ssem, csem = rest
        cid = lax.axis_index("c")
        sid = lax.axis_index("s")
        base = jnp.where(cid == 0, sid * K0, NS * K0 + sid * K1)
        npairs = jnp.where(cid == 0, (NCH0 - 1) // 2, (NCH1 - 1) // 2)
        clast = jnp.where(cid == 0, NCH0 - 1, NCH1 - 1)
        kmine = jnp.where(cid == 0, K0, K1)

        def stage_chunk(c1, s):
            off = pl.multiple_of(base + c1 * CH, CH)
            pltpu.async_copy(src_hbm.at[pl.ds(off, CH)], sidxb.at[s], isem)
            pltpu.async_copy(dst_hbm.at[pl.ds(off, CH)], didxb.at[s], isem)

        def wait_stage(s):
            pltpu.make_async_copy(src_hbm.at[pl.ds(0, CH)], sidxb.at[s],
                                  isem).wait()
            pltpu.make_async_copy(dst_hbm.at[pl.ds(0, CH)], didxb.at[s],
                                  isem).wait()

        def fire_gather(s, jrow, b):
            pltpu.async_copy(tab_hbm.at[sidxb.at[s, jrow]], rows.at[b], gsem)

        def wait_gather(b):
            pltpu.make_async_copy(tab_hbm.at[sidxb.at[0, 0]], rows.at[b],
                                  gsem).wait()

        def wait_scatter():
            pltpu.make_async_copy(rows.at[0], acc.at[didxb.at[0, 0]],
                                  ssem).wait()

        stage_chunk(0, 0)
        wait_stage(0)
        for b in range(NB - 1):
            fire_gather(0, b, b)
        pltpu.sync_copy(znd_hbm.at[pl.ds(sid * rpt, rpt)],
                        acc.at[pl.ds(sid * rpt, rpt)])
        if with_counts:
            pltpu.sync_copy(zn_hbm.at[pl.ds(sid * rpt, rpt)],
                            cacc.at[pl.ds(sid * rpt, rpt)])
            pltpu.sync_copy(ones_hbm, onesv)
        plsc.subcore_barrier()

        def chunk_work(c, h, is_last):
            for j in range(CH):
                b = j % NB
                wait_gather(b)
                pltpu.async_copy(rows.at[b], acc.at[didxb.at[h, j]],
                                 ssem, add=True)
                if with_counts:
                    pltpu.async_copy(onesv, cacc.at[didxb.at[h, j]],
                                     csem, add=True)
                if j == 0:
                    if is_last:
                        @pl.when(c >= 1)
                        def _():
                            wait_scatter()
                    else:
                        @pl.when(c >= 1)
                        def _():
                            wait_scatter()
                        stage_chunk(c + 1, 1 - h)
                else:
                    wait_scatter()
                jn = j + NB - 1
                if jn < CH:
                    fire_gather(h, jn, jn % NB)
                elif not is_last:
                    if jn == CH:
                        wait_stage(1 - h)
                    fire_gather(1 - h, jn - CH, jn % NB)

        def pair_body(c2, carry):
            chunk_work(2 * c2, 0, False)
            chunk_work(2 * c2 + 1, 1, False)
            return carry

        lax.fori_loop(0, npairs, pair_body, 0)
        chunk_work(clast, 0, True)
        wait_scatter()
        if with_counts:
            def cdrain(g, carry):
                pltpu.make_async_copy(onesv, cacc.at[didxb.at[0, 0]],
                                      csem).wait()
                return carry
            lax.fori_loop(0, kmine, cdrain, 0)
        plsc.subcore_barrier()
        pltpu.sync_copy(acc.at[pl.ds(sid * rpt, rpt)],
                        part_out.at[cid, pl.ds(sid * rpt, rpt)])
        if with_counts:
            pltpu.sync_copy(cacc.at[pl.ds(sid * rpt, rpt)],
                            cnt_out.at[cid, pl.ds(sid * rpt, rpt)])

    return pl.kernel(body, out_type=out_type, mesh=mesh, scratch_types=scratch)


def _tc_layer1_body(part_ref, cnt_ref, x_ref, wl_ref, bl_ref, wr_ref,
                    g_ref, b_ref, h_ref):
    agg = part_ref[0] + part_ref[1]
    c = cnt_ref[0] + cnt_ref[1]
    mean = agg * (1.0 / jnp.maximum(c, 1.0))[:, None]
    t = (jnp.dot(mean, wl_ref[...], preferred_element_type=jnp.float32)
         + bl_ref[...]
         + jnp.dot(x_ref[...], wr_ref[...], preferred_element_type=jnp.float32))
    mu = jnp.mean(t, axis=-1, keepdims=True)
    var = jnp.mean((t - mu) ** 2, axis=-1, keepdims=True)
    ln = (t - mu) * lax.rsqrt(var + 1e-5) * g_ref[...] + b_ref[...]
    h_ref[...] = jnp.maximum(ln, 0.0)


def _tc_layer2_body(part_ref, cnt_ref, h_ref, wl_ref, bl_ref, wr_ref, o_ref):
    agg = part_ref[0] + part_ref[1]
    c = cnt_ref[0] + cnt_ref[1]
    mean = agg * (1.0 / jnp.maximum(c, 1.0))[:, None]
    o_ref[...] = (jnp.dot(mean, wl_ref[...], preferred_element_type=jnp.float32)
                  + bl_ref[...]
                  + jnp.dot(h_ref[...], wr_ref[...],
                            preferred_element_type=jnp.float32))


def _make_tc_layer1(N, D, Nacc, R):
    grid = (pl.cdiv(N, R),)
    mat = pl.BlockSpec((D, D), lambda i: (0, 0))
    vec = pl.BlockSpec((1, D), lambda i: (0, 0))
    return pl.pallas_call(
        _tc_layer1_body,
        grid=grid,
        in_specs=[
            pl.BlockSpec((NCS, R, D), lambda i: (0, i, 0)),
            pl.BlockSpec((NCS, R), lambda i: (0, i)),
            pl.BlockSpec((R, D), lambda i: (i, 0)),
            mat, vec, mat, vec, vec,
        ],
        out_specs=pl.BlockSpec((R, D), lambda i: (i, 0)),
        out_shape=jax.ShapeDtypeStruct((N, D), jnp.float32),
    )


def _make_tc_layer2(N, D, Nacc, R):
    grid = (pl.cdiv(N, R),)
    mat = pl.BlockSpec((D, D), lambda i: (0, 0))
    vec = pl.BlockSpec((1, D), lambda i: (0, 0))
    return pl.pallas_call(
        _tc_layer2_body,
        grid=grid,
        in_specs=[
            pl.BlockSpec((NCS, R, D), lambda i: (0, i, 0)),
            pl.BlockSpec((NCS, R), lambda i: (0, i)),
            pl.BlockSpec((R, D), lambda i: (i, 0)),
            mat, vec, mat,
        ],
        out_specs=pl.BlockSpec((R, D), lambda i: (i, 0)),
        out_shape=jax.ShapeDtypeStruct((N, D), jnp.float32),
    )


def kernel(x, edge_index, W1l, b1l, W1r, ln_g, ln_b, W2l, b2l, W2r):
    N, D = x.shape
    E = edge_index.shape[1]
    NW = NCS * NS
    CH = 8
    Gw = 80
    K = pl.cdiv(E, NW * Gw)
    K = ((K + CH - 1) // CH) * CH
    Epad = NW * Gw * K
    K1 = 3 * CH
    K0 = 2 * K - K1
    R = 2048
    nblk = pl.cdiv(N, R)
    Nacc = max(nblk * R, N + 1)
    Nacc = ((Nacc + NS * 8 - 1) // (NS * 8)) * (NS * 8)

    src = edge_index[0]
    dst = edge_index[1]
    pad = Epad - E
    src3 = jnp.concatenate(
        [src, jnp.zeros((pad,), jnp.int32)]).reshape(Epad // Gw, Gw)
    dst3 = jnp.concatenate(
        [dst, jnp.full((pad,), N, jnp.int32)]).reshape(Epad // Gw, Gw)
    zeros_nd = jnp.zeros((Nacc, D), jnp.float32)
    zeros_n = jnp.zeros((Nacc,), jnp.float32)
    ones_g = jnp.ones((Gw,), jnp.float32)

    sc_agg_cnt = _make_sc_agg(K0, K1, Nacc, D, Gw, with_counts=True)
    sc_agg = _make_sc_agg(K0, K1, Nacc, D, Gw, with_counts=False)
    tc1 = _make_tc_layer1(N, D, Nacc, R)
    tc2 = _make_tc_layer2(N, D, Nacc, R)

    part1, cntp = sc_agg_cnt(src3, dst3, x, zeros_nd, zeros_n, ones_g)
    h = tc1(part1, cntp, x, W1l, b1l.reshape(1, D), W1r,
            ln_g.reshape(1, D), ln_b.reshape(1, D))
    (part2,) = sc_agg(src3, dst3, h, zeros_nd, zeros_n, ones_g)
    out = tc2(part2, cntp, h, W2l, b2l.reshape(1, D), W2r)
    return out
